# Initial kernel scaffold; baseline (speedup 1.0000x reference)
import jax
import jax.numpy as jnp
from jax.experimental import pallas as pl


def kernel(x, adj, head, Wg1, Wg2, Wb1, Wb2, r_param, W0, a0, W1, a1, W2, a2):
    raise NotImplementedError("write your pallas kernel here")


# speedup vs baseline: 1.5324x; 1.5324x over previous
import jax
import jax.numpy as jnp
from jax.experimental import pallas as pl
from jax.experimental.pallas import tpu as pltpu

_N = 4096
_NFEAT = 128
_NHID = 64
_NHEADS = 3
_B = 256
_NB = _N // _B


def _leaky(v):
    return jnp.where(v >= 0, v, 0.2 * v)


def _fused_body(x_ref, adj_ref, Wg1_ref, Wg2_ref, Wb1_ref, Wb2_ref, r_ref,
                W_all_ref, a_all_ref, hk_ref, out_ref, h_scr):
    i = pl.program_id(0)
    x_full = x_ref[...]

    @pl.when(i == 0)
    def _init():
        for k in range(_NHEADS):
            h_scr[:, k * _NHID:(k + 1) * _NHID] = jnp.dot(
                x_full, W_all_ref[k], preferred_element_type=jnp.float32)

    adj_blk = adj_ref[...]
    x_blk = x_ref[pl.ds(i * _B, _B), :]

    s = jnp.sum(jnp.abs(adj_blk), axis=1, keepdims=True)
    nb = jnp.dot(adj_blk, x_full, preferred_element_type=jnp.float32)
    nb = nb / jnp.maximum(s, 1e-12)

    gamma = _leaky(jnp.dot(x_blk, Wg1_ref[...], preferred_element_type=jnp.float32)
                   + jnp.dot(nb, Wg2_ref[...], preferred_element_type=jnp.float32)) + 1.0
    beta = _leaky(jnp.dot(x_blk, Wb1_ref[...], preferred_element_type=jnp.float32)
                  + jnp.dot(nb, Wb2_ref[...], preferred_element_type=jnp.float32))
    r_v = gamma * r_ref[...] + beta
    out_ref[...] = x_blk + r_v - nb

    row_ids = jax.lax.broadcasted_iota(jnp.int32, (_B, _N), 0) + i * _B
    col_ids = jax.lax.broadcasted_iota(jnp.int32, (_B, _N), 1)
    mask = (adj_blk > 0) | (row_ids == col_ids)

    for k in range(_NHEADS):
        h = h_scr[:, k * _NHID:(k + 1) * _NHID]
        h_blk = h_scr[pl.ds(i * _B, _B), k * _NHID:(k + 1) * _NHID]
        a2d = a_all_ref[k:k + 1, :]
        a_src = a2d[:, :_NHID]
        a_dst = a2d[:, _NHID:]
        f_src = jax.lax.dot_general(h_blk, a_src, (((1,), (1,)), ((), ())),
                                    preferred_element_type=jnp.float32)
        f_dst = jax.lax.dot_general(a_dst, h, (((1,), (1,)), ((), ())),
                                    preferred_element_type=jnp.float32)
        ea_src = jnp.exp(-f_src)
        ea_dst = jnp.exp(-f_dst)
        eb_src = jnp.exp(-0.2 * f_src)
        eb_dst = jnp.exp(-0.2 * f_dst)
        s_mat = f_src + f_dst
        e = jnp.where(mask,
                      jnp.where(s_mat >= 0, ea_src * ea_dst, eb_src * eb_dst),
                      0.0)
        e_rowsum = jnp.sum(e, axis=1, keepdims=True)
        hp = jnp.dot(e, h, preferred_element_type=jnp.float32) / e_rowsum
        hk_ref[:, k * _NHID:(k + 1) * _NHID] = jnp.where(
            hp > 0, hp, jnp.exp(jnp.minimum(hp, 0.0)) - 1.0)


def kernel(x, adj, head, Wg1, Wg2, Wb1, Wb2, r_param, W0, a0, W1, a1, W2, a2):
    del head
    W_all = jnp.stack([W0, W1, W2])
    a_all = jnp.concatenate([a0, a1, a2], axis=0)

    h_k, output = pl.pallas_call(
        _fused_body,
        grid=(_NB,),
        in_specs=[
            pl.BlockSpec((_N, _NFEAT), lambda i: (0, 0)),
            pl.BlockSpec((_B, _N), lambda i: (i, 0)),
            pl.BlockSpec((_NFEAT, _NFEAT), lambda i: (0, 0)),
            pl.BlockSpec((_NFEAT, _NFEAT), lambda i: (0, 0)),
            pl.BlockSpec((_NFEAT, _NFEAT), lambda i: (0, 0)),
            pl.BlockSpec((_NFEAT, _NFEAT), lambda i: (0, 0)),
            pl.BlockSpec((1, _NFEAT), lambda i: (0, 0)),
            pl.BlockSpec((_NHEADS, _NFEAT, _NHID), lambda i: (0, 0, 0)),
            pl.BlockSpec((_NHEADS, 2 * _NHID), lambda i: (0, 0)),
        ],
        out_specs=[
            pl.BlockSpec((_B, _NHEADS * _NHID), lambda i: (i, 0)),
            pl.BlockSpec((_B, _NFEAT), lambda i: (i, 0)),
        ],
        out_shape=[
            jax.ShapeDtypeStruct((_N, _NHEADS * _NHID), jnp.float32),
            jax.ShapeDtypeStruct((_N, _NFEAT), jnp.float32),
        ],
        scratch_shapes=[pltpu.VMEM((_N, _NHEADS * _NHID), jnp.float32)],
    )(x, adj, Wg1, Wg2, Wb1, Wb2, r_param, W_all, a_all)
    return (h_k, output)
